# Initial kernel scaffold; baseline (speedup 1.0000x reference)
import jax
import jax.numpy as jnp
from jax.experimental import pallas as pl


def kernel(xyz, points):
    raise NotImplementedError("write your pallas kernel here")


# speedup vs baseline: 5.1200x; 5.1200x over previous
import functools

import jax
import jax.numpy as jnp
from jax.experimental import pallas as pl

_N = 4096
_B = 4
_K = 32
_QB = 128
_NQB = _N // _QB


def _walk_body(has_noise, n_keys, xk_ref, xq_ref, c_ref, u_ref, *rest):
    if has_noise:
        nz_ref, out_ref = rest
    else:
        (out_ref,) = rest
    xk = xk_ref[0]
    xq = xq_ref[0]
    sqk = jnp.sum(xk * xk, axis=1)[None, :]
    sqq = jnp.sum(xq * xq, axis=1, keepdims=True)
    dot = jax.lax.dot_general(
        xq, xk, (((1,), (1,)), ((), ())),
        preferred_element_type=jnp.float32)
    d2 = sqq + sqk - 2.0 * dot

    iota = jax.lax.broadcasted_iota(jnp.int32, (_QB, n_keys), 1)
    c = c_ref[0]
    masked = d2
    sel = jnp.zeros((_QB, 1), jnp.int32)
    for t in range(_K):
        m = jnp.min(masked, axis=1, keepdims=True)
        cand = jnp.where(masked == m, iota, n_keys)
        amin = jnp.min(cand, axis=1, keepdims=True)
        sel = jnp.where(c == t, amin, sel)
        masked = jnp.where(iota == amin, jnp.float32(jnp.inf), masked)

    onehot = (iota == sel).astype(jnp.float32)
    nbr = jax.lax.dot_general(
        onehot, xk, (((1,), (0,)), ((), ())),
        precision=jax.lax.Precision.HIGHEST,
        preferred_element_type=jnp.float32)
    u = u_ref[0]
    new = xq + u * (nbr - xq)
    if has_noise:
        new = new + nz_ref[0]
    out_ref[0] = new


def _random_walk(x, noise, key):
    b, n, c_dim = x.shape
    k1, k2, k3 = jax.random.split(key, 3)
    choice = jax.random.randint(k1, (b, n, 1), 0, _K)
    u = jax.random.uniform(k2, (b, n, 1), dtype=x.dtype)
    nz = noise * jax.random.normal(k3, x.shape, dtype=x.dtype) if noise > 0.0 else None

    c3 = choice[..., 0].astype(jnp.int32).reshape(b * _NQB, _QB, 1)
    u3 = u[..., 0].reshape(b * _NQB, _QB, 1)
    in_specs = [
        pl.BlockSpec((1, n, c_dim), lambda bi, qi: (bi, 0, 0)),
        pl.BlockSpec((1, _QB, c_dim), lambda bi, qi: (bi, qi, 0)),
        pl.BlockSpec((1, _QB, 1), lambda bi, qi: (bi * _NQB + qi, 0, 0)),
        pl.BlockSpec((1, _QB, 1), lambda bi, qi: (bi * _NQB + qi, 0, 0)),
    ]
    args = [x, x, c3, u3]
    if nz is not None:
        in_specs.append(pl.BlockSpec((1, _QB, c_dim), lambda bi, qi: (bi, qi, 0)))
        args.append(nz)
    new = pl.pallas_call(
        functools.partial(_walk_body, nz is not None, n),
        grid=(b, _NQB),
        in_specs=in_specs,
        out_specs=pl.BlockSpec((1, _QB, c_dim), lambda bi, qi: (bi, qi, 0)),
        out_shape=jax.ShapeDtypeStruct((b, n, c_dim), jnp.float32),
    )(*args)
    return jnp.concatenate([x, new], axis=1)


def kernel(xyz, points):
    x1 = jnp.transpose(xyz[..., 0], (0, 2, 1))
    x2 = jnp.transpose(points[..., 0], (0, 2, 1))
    out1 = _random_walk(x1, 0.05, jax.random.key(1))
    out2 = _random_walk(x2, 0.0, jax.random.key(2))
    return (out1, out2)


# speedup vs baseline: 6.5480x; 1.2789x over previous
import functools

import jax
import jax.numpy as jnp
from jax.experimental import pallas as pl

_N = 4096
_B = 4
_K = 32
_QB = 128
_NQB = _N // _QB


def _walk_body(has_noise, n_keys, xk_ref, xq_ref, c_ref, u_ref, *rest):
    if has_noise:
        nz_ref, out_ref = rest
    else:
        (out_ref,) = rest
    xk = xk_ref[0]
    xq = xq_ref[0]
    sqk = jnp.sum(xk * xk, axis=1)[None, :]
    sqq = jnp.sum(xq * xq, axis=1, keepdims=True)
    dot = jax.lax.dot_general(
        xq, xk, (((1,), (1,)), ((), ())),
        preferred_element_type=jnp.float32)
    d2 = sqq + sqk - 2.0 * dot

    iota = jax.lax.broadcasted_iota(
        jnp.int32, (_QB, n_keys), 1).astype(jnp.float32)
    c = c_ref[0]
    masked = d2
    sel = jnp.zeros((_QB, 1), jnp.float32)
    big = jnp.float32(n_keys)
    for t in range(_K):
        m = jnp.min(masked, axis=1, keepdims=True)
        tie = jnp.where(masked == m, iota, big)
        amin = jnp.min(tie, axis=1, keepdims=True)
        sel = jnp.where(c == t, amin, sel)
        masked = jnp.where(tie == amin, jnp.float32(jnp.inf), masked)

    onehot = (iota == sel).astype(jnp.float32)
    nbr = jax.lax.dot_general(
        onehot, xk, (((1,), (0,)), ((), ())),
        precision=jax.lax.Precision.HIGHEST,
        preferred_element_type=jnp.float32)
    u = u_ref[0]
    new = xq + u * (nbr - xq)
    if has_noise:
        new = new + nz_ref[0]
    out_ref[0] = new


def _random_walk(x, noise, key):
    b, n, c_dim = x.shape
    k1, k2, k3 = jax.random.split(key, 3)
    choice = jax.random.randint(k1, (b, n, 1), 0, _K)
    u = jax.random.uniform(k2, (b, n, 1), dtype=x.dtype)
    nz = noise * jax.random.normal(k3, x.shape, dtype=x.dtype) if noise > 0.0 else None

    c3 = choice[..., 0].astype(jnp.int32).reshape(b * _NQB, _QB, 1)
    u3 = u[..., 0].reshape(b * _NQB, _QB, 1)
    in_specs = [
        pl.BlockSpec((1, n, c_dim), lambda bi, qi: (bi, 0, 0)),
        pl.BlockSpec((1, _QB, c_dim), lambda bi, qi: (bi, qi, 0)),
        pl.BlockSpec((1, _QB, 1), lambda bi, qi: (bi * _NQB + qi, 0, 0)),
        pl.BlockSpec((1, _QB, 1), lambda bi, qi: (bi * _NQB + qi, 0, 0)),
    ]
    args = [x, x, c3, u3]
    if nz is not None:
        in_specs.append(pl.BlockSpec((1, _QB, c_dim), lambda bi, qi: (bi, qi, 0)))
        args.append(nz)
    new = pl.pallas_call(
        functools.partial(_walk_body, nz is not None, n),
        grid=(b, _NQB),
        in_specs=in_specs,
        out_specs=pl.BlockSpec((1, _QB, c_dim), lambda bi, qi: (bi, qi, 0)),
        out_shape=jax.ShapeDtypeStruct((b, n, c_dim), jnp.float32),
    )(*args)
    return jnp.concatenate([x, new], axis=1)


def kernel(xyz, points):
    x1 = jnp.transpose(xyz[..., 0], (0, 2, 1))
    x2 = jnp.transpose(points[..., 0], (0, 2, 1))
    out1 = _random_walk(x1, 0.05, jax.random.key(1))
    out2 = _random_walk(x2, 0.0, jax.random.key(2))
    return (out1, out2)
